# Initial kernel scaffold; baseline (speedup 1.0000x reference)
import jax
import jax.numpy as jnp
from jax.experimental import pallas as pl


def kernel(u, i, edges, user_table, item_table, W1, b1, W2, b2):
    raise NotImplementedError("write your pallas kernel here")


# speedup vs baseline: 11.2043x; 11.2043x over previous
import functools
import jax
import jax.numpy as jnp
from jax import lax
from jax.experimental import pallas as pl
from jax.experimental.pallas import tpu as pltpu
from jax.experimental.pallas import tpu_sc as plsc

NI = 10000
D = 128
E = 320000
B = 4096
NC = 2
NS = 16
NW = NC * NS
EPT = E // NW
CH = 80
NCHUNK = EPT // CH
RPT = 632
RPT_LAST = NI - 15 * RPT
BPT = B // NW

_MESH = plsc.VectorSubcoreMesh(
    core_axis_name="c", subcore_axis_name="s", num_cores=NC, num_subcores=NS)


def _wid():
  return lax.axis_index("c") * NS + lax.axis_index("s")


def _copy_rows(src, dst, s):
  base = pl.multiple_of(s * RPT, 8)

  @pl.when(s < NS - 1)
  def _():
    pltpu.sync_copy(src.at[pl.ds(base, RPT)], dst.at[pl.ds(base, RPT)])

  @pl.when(s == NS - 1)
  def _():
    pltpu.sync_copy(src.at[pl.ds(base, RPT_LAST)], dst.at[pl.ds(base, RPT_LAST)])


@functools.partial(
    pl.kernel,
    out_type=[jax.ShapeDtypeStruct((NI, D), jnp.float32),
              jax.ShapeDtypeStruct((NI, D), jnp.float32)],
    mesh=_MESH,
    compiler_params=pltpu.CompilerParams(needs_layout_passes=False),
    scratch_types=[
        pltpu.VMEM((CH,), jnp.int32),
        pltpu.VMEM((CH, D), jnp.float32),
        pltpu.VMEM_SHARED((NI, D), jnp.float32),
    ],
)
def _deg_kernel(dst_hbm, zeros16_hbm, ones_hbm, deg0_hbm, deg1_hbm,
                idx_v, ones_v, acc_sh):
  c = lax.axis_index("c")
  s = lax.axis_index("s")
  wid = c * NS + s
  _copy_rows(zeros16_hbm, acc_sh, s)
  pltpu.sync_copy(ones_hbm, ones_v)
  plsc.subcore_barrier()

  def chunk(g, carry):
    base = pl.multiple_of(wid * EPT + g * CH, 8)
    pltpu.sync_copy(dst_hbm.at[pl.ds(base, CH)], idx_v)
    pltpu.sync_copy(ones_v, acc_sh.at[idx_v], add=True)
    return carry

  lax.fori_loop(0, NCHUNK, chunk, 0)
  plsc.subcore_barrier()

  @pl.when(c == 0)
  def _():
    _copy_rows(acc_sh, deg0_hbm, s)

  @pl.when(c == 1)
  def _():
    _copy_rows(acc_sh, deg1_hbm, s)


@functools.partial(
    pl.kernel,
    out_type=[jax.ShapeDtypeStruct((NI, D), jnp.float32),
              jax.ShapeDtypeStruct((NI, D), jnp.float32)],
    mesh=_MESH,
    compiler_params=pltpu.CompilerParams(needs_layout_passes=False),
    scratch_types=[
        pltpu.VMEM((CH,), jnp.int32),
        pltpu.VMEM((CH,), jnp.int32),
        pltpu.VMEM((CH, D), jnp.float32),
        pltpu.VMEM_SHARED((NI, D), jnp.float32),
        pltpu.SemaphoreType.DMA,
    ],
)
def _prop_kernel(y_hbm, src_hbm, dst_hbm, zeros_hbm, s0_hbm, s1_hbm,
                 sidx_v, didx_v, rows_v, acc_sh, sem):
  c = lax.axis_index("c")
  s = lax.axis_index("s")
  wid = c * NS + s
  _copy_rows(zeros_hbm, acc_sh, s)
  plsc.subcore_barrier()

  def chunk(g, carry):
    base = pl.multiple_of(wid * EPT + g * CH, 8)
    pltpu.sync_copy(src_hbm.at[pl.ds(base, CH)], sidx_v)
    pltpu.sync_copy(dst_hbm.at[pl.ds(base, CH)], didx_v)
    pltpu.async_copy(y_hbm.at[sidx_v], rows_v, sem).wait()
    pltpu.sync_copy(rows_v, acc_sh.at[didx_v], add=True)
    return carry

  lax.fori_loop(0, NCHUNK, chunk, 0)
  plsc.subcore_barrier()

  @pl.when(c == 0)
  def _():
    _copy_rows(acc_sh, s0_hbm, s)

  @pl.when(c == 1)
  def _():
    _copy_rows(acc_sh, s1_hbm, s)


_RB = 1000


def _tc1_body(item_ref, deg0_ref, deg1_ref, w1_ref, y1_ref, dinv_ref):
  x = item_ref[...]
  ss = jnp.sum(x * x, axis=1, keepdims=True)
  n = jnp.sqrt(ss)
  scale = jnp.where(n > 1.0, 1.0 / (n + 1e-7), 1.0)
  deg = deg0_ref[:, 0:1] + deg1_ref[:, 0:1] + 1.0
  dinv = lax.rsqrt(deg)
  xs = x * (scale * dinv)
  y1_ref[...] = jnp.dot(xs, w1_ref[...], preferred_element_type=jnp.float32)
  dinv_ref[...] = jnp.broadcast_to(dinv, (_RB, D))


def _tc1(item_table, deg0, deg1, w1):
  return pl.pallas_call(
      _tc1_body,
      grid=(NI // _RB,),
      in_specs=[
          pl.BlockSpec((_RB, D), lambda g: (g, 0)),
          pl.BlockSpec((_RB, D), lambda g: (g, 0)),
          pl.BlockSpec((_RB, D), lambda g: (g, 0)),
          pl.BlockSpec((D, D), lambda g: (0, 0)),
      ],
      out_specs=[
          pl.BlockSpec((_RB, D), lambda g: (g, 0)),
          pl.BlockSpec((_RB, D), lambda g: (g, 0)),
      ],
      out_shape=[jax.ShapeDtypeStruct((NI, D), jnp.float32),
                 jax.ShapeDtypeStruct((NI, D), jnp.float32)],
  )(item_table, deg0, deg1, w1)


def _tc2_body(s0_ref, s1_ref, y1_ref, dinv_ref, w2_ref, b1_ref, y2_ref):
  dinv = dinv_ref[:, 0:1]
  z = dinv * (s0_ref[...] + s1_ref[...] + y1_ref[...]) + b1_ref[...]
  z = jnp.maximum(z, 0.0)
  y2_ref[...] = jnp.dot(z * dinv, w2_ref[...],
                        preferred_element_type=jnp.float32)


def _tc2(s0, s1, y1, dinv16, w2, b1):
  return pl.pallas_call(
      _tc2_body,
      grid=(NI // _RB,),
      in_specs=[
          pl.BlockSpec((_RB, D), lambda g: (g, 0)),
          pl.BlockSpec((_RB, D), lambda g: (g, 0)),
          pl.BlockSpec((_RB, D), lambda g: (g, 0)),
          pl.BlockSpec((_RB, D), lambda g: (g, 0)),
          pl.BlockSpec((D, D), lambda g: (0, 0)),
          pl.BlockSpec((1, D), lambda g: (0, 0)),
      ],
      out_specs=pl.BlockSpec((_RB, D), lambda g: (g, 0)),
      out_shape=jax.ShapeDtypeStruct((NI, D), jnp.float32),
  )(s0, s1, y1, dinv16, w2, b1)


def _rsqrt_nr(x):
  i = plsc.bitcast(x, jnp.int32)
  i = 0x5F3759DF - lax.shift_right_arithmetic(i, 1)
  y = plsc.bitcast(i, jnp.float32)
  for _ in range(3):
    y = y * (1.5 - 0.5 * x * y * y)
  return y


@functools.partial(
    pl.kernel,
    out_type=jax.ShapeDtypeStruct((B,), jnp.float32),
    mesh=_MESH,
    compiler_params=pltpu.CompilerParams(needs_layout_passes=False),
    scratch_types=[
        pltpu.VMEM((BPT,), jnp.int32),
        pltpu.VMEM((BPT,), jnp.int32),
        pltpu.VMEM((BPT, D), jnp.float32),
        pltpu.VMEM((BPT, D), jnp.float32),
        pltpu.VMEM((BPT, D), jnp.float32),
        pltpu.VMEM((BPT, D), jnp.float32),
        pltpu.VMEM((BPT, D), jnp.float32),
        pltpu.VMEM((D,), jnp.float32),
        pltpu.VMEM((BPT,), jnp.float32),
        pltpu.VMEM((256,), jnp.float32),
        pltpu.VMEM((256,), jnp.float32),
        pltpu.SemaphoreType.DMA,
    ],
)
def _score_kernel(u_hbm, i_hbm, utab_hbm, t0_hbm, t1_hbm, y2_hbm, dinv_hbm,
                  b2_hbm, out_hbm,
                  uidx_v, iidx_v, urows_v, t0_v, t1_v, y2_v, dv_v, b2_v,
                  res_v, ssm_v, dotm_v, sem):
  wid = _wid()
  base = pl.multiple_of(wid * BPT, 8)
  pltpu.sync_copy(u_hbm.at[pl.ds(base, BPT)], uidx_v)
  pltpu.sync_copy(i_hbm.at[pl.ds(base, BPT)], iidx_v)
  pltpu.sync_copy(b2_hbm, b2_v)
  pltpu.async_copy(utab_hbm.at[uidx_v], urows_v, sem).wait()
  pltpu.async_copy(t0_hbm.at[iidx_v], t0_v, sem).wait()
  pltpu.async_copy(t1_hbm.at[iidx_v], t1_v, sem).wait()
  pltpu.async_copy(y2_hbm.at[iidx_v], y2_v, sem).wait()
  pltpu.async_copy(dinv_hbm.at[iidx_v], dv_v, sem).wait()

  lane = lax.broadcasted_iota(jnp.int32, (16,), 0)

  def group(grp, carry):
    for j in range(16):
      r = grp * 16 + j
      dinv = dv_v[r, pl.ds(0, 16)]
      acc_ss = jnp.zeros((16,), jnp.float32)
      acc_dot = jnp.zeros((16,), jnp.float32)
      for k in range(D // 16):
        sl = pl.ds(16 * k, 16)
        uc = urows_v[r, sl]
        ic = dinv * (t0_v[r, sl] + t1_v[r, sl] + y2_v[r, sl]) + b2_v[sl]
        acc_ss = acc_ss + uc * uc
        acc_dot = acc_dot + uc * ic
      ssm_v[pl.ds(16 * j, 16)] = acc_ss
      dotm_v[pl.ds(16 * j, 16)] = acc_dot
    ss = jnp.zeros((16,), jnp.float32)
    dot = jnp.zeros((16,), jnp.float32)
    for k in range(16):
      col = lane * 16 + k
      ss = ss + plsc.load_gather(ssm_v, [col])
      dot = dot + plsc.load_gather(dotm_v, [col])
    rinv = _rsqrt_nr(jnp.maximum(ss, 1e-12))
    n = ss * rinv
    scale = jnp.where(n > 1.0, 1.0 / (n + 1e-7), 1.0)
    uv = scale * dot
    sig = 1.0 / (1.0 + jnp.exp(-uv))
    res_v[pl.ds(pl.multiple_of(grp * 16, 16), 16)] = sig
    return carry

  lax.fori_loop(0, BPT // 16, group, 0)
  pltpu.sync_copy(res_v, out_hbm.at[pl.ds(base, BPT)])


def kernel(u, i, edges, user_table, item_table, W1, b1, W2, b2):
  u = u.astype(jnp.int32)
  i = i.astype(jnp.int32)
  edges = edges.astype(jnp.int32)
  src = edges[0]
  dst = edges[1]
  zeros128 = jnp.zeros((NI, D), jnp.float32)
  ones128 = jnp.ones((CH, D), jnp.float32)

  deg0, deg1 = _deg_kernel(dst, zeros128, ones128)
  y1, dinv16 = _tc1(item_table, deg0, deg1, W1)
  s0, s1 = _prop_kernel(y1, src, dst, zeros128)
  y2 = _tc2(s0, s1, y1, dinv16, W2, b1.reshape(1, D))
  t0, t1 = _prop_kernel(y2, src, dst, zeros128)
  out = _score_kernel(u, i, user_table, t0, t1, y2, dinv16, b2)
  return out


# speedup vs baseline: 23.2458x; 2.0747x over previous
import functools
import jax
import jax.numpy as jnp
from jax import lax
from jax.experimental import pallas as pl
from jax.experimental.pallas import tpu as pltpu
from jax.experimental.pallas import tpu_sc as plsc

NI = 10000
D = 128
E = 320000
B = 4096
NC = 2
NS = 16
NW = NC * NS
EPT = E // NW
CH = 80
NCHUNK = EPT // CH
RPT = 632
RPT_LAST = NI - 15 * RPT
BPT = B // NW

_MESH = plsc.VectorSubcoreMesh(
    core_axis_name="c", subcore_axis_name="s", num_cores=NC, num_subcores=NS)


def _wid():
  return lax.axis_index("c") * NS + lax.axis_index("s")


def _copy_rows(src, dst, s):
  base = pl.multiple_of(s * RPT, 8)

  @pl.when(s < NS - 1)
  def _():
    pltpu.sync_copy(src.at[pl.ds(base, RPT)], dst.at[pl.ds(base, RPT)])

  @pl.when(s == NS - 1)
  def _():
    pltpu.sync_copy(src.at[pl.ds(base, RPT_LAST)], dst.at[pl.ds(base, RPT_LAST)])


@functools.partial(
    pl.kernel,
    out_type=[jax.ShapeDtypeStruct((NI, D), jnp.float32),
              jax.ShapeDtypeStruct((NI, D), jnp.float32)],
    mesh=_MESH,
    compiler_params=pltpu.CompilerParams(needs_layout_passes=False),
    scratch_types=[
        pltpu.VMEM((CH,), jnp.int32),
        pltpu.VMEM((CH, D), jnp.float32),
        pltpu.VMEM_SHARED((NI, D), jnp.float32),
    ],
)
def _deg_kernel(dst_hbm, zeros16_hbm, ones_hbm, deg0_hbm, deg1_hbm,
                idx_v, ones_v, acc_sh):
  c = lax.axis_index("c")
  s = lax.axis_index("s")
  wid = c * NS + s
  _copy_rows(zeros16_hbm, acc_sh, s)
  pltpu.sync_copy(ones_hbm, ones_v)
  plsc.subcore_barrier()

  def chunk(g, carry):
    base = pl.multiple_of(wid * EPT + g * CH, 8)
    pltpu.sync_copy(dst_hbm.at[pl.ds(base, CH)], idx_v)
    pltpu.sync_copy(ones_v, acc_sh.at[idx_v], add=True)
    return carry

  lax.fori_loop(0, NCHUNK, chunk, 0)
  plsc.subcore_barrier()

  @pl.when(c == 0)
  def _():
    _copy_rows(acc_sh, deg0_hbm, s)

  @pl.when(c == 1)
  def _():
    _copy_rows(acc_sh, deg1_hbm, s)


NRING = 3
NFULL = (NCHUNK - 2) // NRING


@functools.partial(
    pl.kernel,
    out_type=[jax.ShapeDtypeStruct((NI, D), jnp.float32),
              jax.ShapeDtypeStruct((NI, D), jnp.float32)],
    mesh=_MESH,
    compiler_params=pltpu.CompilerParams(needs_layout_passes=False),
    scratch_types=[
        pltpu.VMEM((EPT,), jnp.int32),
        pltpu.VMEM((EPT,), jnp.int32),
        pltpu.VMEM((NRING, CH, D), jnp.float32),
        pltpu.VMEM_SHARED((NI, D), jnp.float32),
        pltpu.SemaphoreType.DMA,
        pltpu.SemaphoreType.DMA,
    ],
)
def _prop_kernel(y_hbm, src_hbm, dst_hbm, zeros_hbm, s0_hbm, s1_hbm,
                 sidx_v, didx_v, rows_v, acc_sh, gsem, ssem):
  c = lax.axis_index("c")
  s = lax.axis_index("s")
  wid = c * NS + s
  _copy_rows(zeros_hbm, acc_sh, s)
  base = pl.multiple_of(wid * EPT, 8)
  pltpu.sync_copy(src_hbm.at[pl.ds(base, EPT)], sidx_v)
  pltpu.sync_copy(dst_hbm.at[pl.ds(base, EPT)], didx_v)
  plsc.subcore_barrier()

  def chunk_slice(g):
    return pl.ds(pl.multiple_of(g * CH, 8), CH)

  def fire_gather(g, b):
    return pltpu.async_copy(y_hbm.at[sidx_v.at[chunk_slice(g)]],
                            rows_v.at[b], gsem)

  def wait_one_gather():
    pltpu.make_async_copy(zeros_hbm.at[pl.ds(0, CH)], rows_v.at[0],
                          gsem).wait()

  def fire_scatter(g, b):
    pltpu.async_copy(rows_v.at[b], acc_sh.at[didx_v.at[chunk_slice(g)]],
                     ssem, add=True)

  def drain_scatters(k):
    for _ in range(k):
      pltpu.make_async_copy(zeros_hbm.at[pl.ds(0, CH)], rows_v.at[0],
                            ssem).wait()

  def ring_round(k, carry):
    for j in range(NRING):
      g = k * NRING + j

      @pl.when(k > 0)
      def _():
        drain_scatters(1)
      fire_gather(g, j)
      if j == NRING - 1:
        wait_one_gather()
        fire_scatter(g - 2, (j + 1) % NRING)
      else:
        @pl.when(k > 0)
        def _():
          wait_one_gather()
          fire_scatter(g - 2, (j + 1) % NRING)
    return carry

  lax.fori_loop(0, NFULL, ring_round, 0)
  gl = NFULL * NRING
  wait_one_gather()
  fire_scatter(gl - 2, (gl - 2) % NRING)
  wait_one_gather()
  fire_scatter(gl - 1, (gl - 1) % NRING)
  drain_scatters(NRING)
  fire_gather(gl, gl % NRING)
  fire_gather(gl + 1, (gl + 1) % NRING)
  wait_one_gather()
  fire_scatter(gl, gl % NRING)
  wait_one_gather()
  fire_scatter(gl + 1, (gl + 1) % NRING)
  drain_scatters(2)
  plsc.subcore_barrier()

  @pl.when(c == 0)
  def _():
    _copy_rows(acc_sh, s0_hbm, s)

  @pl.when(c == 1)
  def _():
    _copy_rows(acc_sh, s1_hbm, s)


_RB = 1000


def _tc1_body(item_ref, deg0_ref, deg1_ref, w1_ref, y1_ref, dinv_ref):
  x = item_ref[...]
  ss = jnp.sum(x * x, axis=1, keepdims=True)
  n = jnp.sqrt(ss)
  scale = jnp.where(n > 1.0, 1.0 / (n + 1e-7), 1.0)
  deg = deg0_ref[:, 0:1] + deg1_ref[:, 0:1] + 1.0
  dinv = lax.rsqrt(deg)
  xs = x * (scale * dinv)
  y1_ref[...] = jnp.dot(xs, w1_ref[...], preferred_element_type=jnp.float32)
  dinv_ref[...] = jnp.broadcast_to(dinv, (_RB, D))


def _tc1(item_table, deg0, deg1, w1):
  return pl.pallas_call(
      _tc1_body,
      grid=(NI // _RB,),
      in_specs=[
          pl.BlockSpec((_RB, D), lambda g: (g, 0)),
          pl.BlockSpec((_RB, D), lambda g: (g, 0)),
          pl.BlockSpec((_RB, D), lambda g: (g, 0)),
          pl.BlockSpec((D, D), lambda g: (0, 0)),
      ],
      out_specs=[
          pl.BlockSpec((_RB, D), lambda g: (g, 0)),
          pl.BlockSpec((_RB, D), lambda g: (g, 0)),
      ],
      out_shape=[jax.ShapeDtypeStruct((NI, D), jnp.float32),
                 jax.ShapeDtypeStruct((NI, D), jnp.float32)],
  )(item_table, deg0, deg1, w1)


def _tc2_body(s0_ref, s1_ref, y1_ref, dinv_ref, w2_ref, b1_ref, y2_ref):
  dinv = dinv_ref[:, 0:1]
  z = dinv * (s0_ref[...] + s1_ref[...] + y1_ref[...]) + b1_ref[...]
  z = jnp.maximum(z, 0.0)
  y2_ref[...] = jnp.dot(z * dinv, w2_ref[...],
                        preferred_element_type=jnp.float32)


def _tc2(s0, s1, y1, dinv16, w2, b1):
  return pl.pallas_call(
      _tc2_body,
      grid=(NI // _RB,),
      in_specs=[
          pl.BlockSpec((_RB, D), lambda g: (g, 0)),
          pl.BlockSpec((_RB, D), lambda g: (g, 0)),
          pl.BlockSpec((_RB, D), lambda g: (g, 0)),
          pl.BlockSpec((_RB, D), lambda g: (g, 0)),
          pl.BlockSpec((D, D), lambda g: (0, 0)),
          pl.BlockSpec((1, D), lambda g: (0, 0)),
      ],
      out_specs=pl.BlockSpec((_RB, D), lambda g: (g, 0)),
      out_shape=jax.ShapeDtypeStruct((NI, D), jnp.float32),
  )(s0, s1, y1, dinv16, w2, b1)


def _rsqrt_nr(x):
  i = plsc.bitcast(x, jnp.int32)
  i = 0x5F3759DF - lax.shift_right_arithmetic(i, 1)
  y = plsc.bitcast(i, jnp.float32)
  for _ in range(3):
    y = y * (1.5 - 0.5 * x * y * y)
  return y


@functools.partial(
    pl.kernel,
    out_type=jax.ShapeDtypeStruct((B,), jnp.float32),
    mesh=_MESH,
    compiler_params=pltpu.CompilerParams(needs_layout_passes=False),
    scratch_types=[
        pltpu.VMEM((BPT,), jnp.int32),
        pltpu.VMEM((BPT,), jnp.int32),
        pltpu.VMEM((BPT, D), jnp.float32),
        pltpu.VMEM((BPT, D), jnp.float32),
        pltpu.VMEM((BPT, D), jnp.float32),
        pltpu.VMEM((BPT, D), jnp.float32),
        pltpu.VMEM((BPT, D), jnp.float32),
        pltpu.VMEM((D,), jnp.float32),
        pltpu.VMEM((BPT,), jnp.float32),
        pltpu.VMEM((256,), jnp.float32),
        pltpu.VMEM((256,), jnp.float32),
        pltpu.SemaphoreType.DMA,
    ],
)
def _score_kernel(u_hbm, i_hbm, utab_hbm, t0_hbm, t1_hbm, y2_hbm, dinv_hbm,
                  b2_hbm, out_hbm,
                  uidx_v, iidx_v, urows_v, t0_v, t1_v, y2_v, dv_v, b2_v,
                  res_v, ssm_v, dotm_v, sem):
  wid = _wid()
  base = pl.multiple_of(wid * BPT, 8)
  pltpu.sync_copy(u_hbm.at[pl.ds(base, BPT)], uidx_v)
  pltpu.sync_copy(i_hbm.at[pl.ds(base, BPT)], iidx_v)
  pltpu.sync_copy(b2_hbm, b2_v)
  pltpu.async_copy(utab_hbm.at[uidx_v], urows_v, sem).wait()
  pltpu.async_copy(t0_hbm.at[iidx_v], t0_v, sem).wait()
  pltpu.async_copy(t1_hbm.at[iidx_v], t1_v, sem).wait()
  pltpu.async_copy(y2_hbm.at[iidx_v], y2_v, sem).wait()
  pltpu.async_copy(dinv_hbm.at[iidx_v], dv_v, sem).wait()

  lane = lax.broadcasted_iota(jnp.int32, (16,), 0)

  def group(grp, carry):
    for j in range(16):
      r = grp * 16 + j
      dinv = dv_v[r, pl.ds(0, 16)]
      acc_ss = jnp.zeros((16,), jnp.float32)
      acc_dot = jnp.zeros((16,), jnp.float32)
      for k in range(D // 16):
        sl = pl.ds(16 * k, 16)
        uc = urows_v[r, sl]
        ic = dinv * (t0_v[r, sl] + t1_v[r, sl] + y2_v[r, sl]) + b2_v[sl]
        acc_ss = acc_ss + uc * uc
        acc_dot = acc_dot + uc * ic
      ssm_v[pl.ds(16 * j, 16)] = acc_ss
      dotm_v[pl.ds(16 * j, 16)] = acc_dot
    ss = jnp.zeros((16,), jnp.float32)
    dot = jnp.zeros((16,), jnp.float32)
    for k in range(16):
      col = lane * 16 + k
      ss = ss + plsc.load_gather(ssm_v, [col])
      dot = dot + plsc.load_gather(dotm_v, [col])
    rinv = _rsqrt_nr(jnp.maximum(ss, 1e-12))
    n = ss * rinv
    scale = jnp.where(n > 1.0, 1.0 / (n + 1e-7), 1.0)
    uv = scale * dot
    sig = 1.0 / (1.0 + jnp.exp(-uv))
    res_v[pl.ds(pl.multiple_of(grp * 16, 16), 16)] = sig
    return carry

  lax.fori_loop(0, BPT // 16, group, 0)
  pltpu.sync_copy(res_v, out_hbm.at[pl.ds(base, BPT)])


def kernel(u, i, edges, user_table, item_table, W1, b1, W2, b2):
  u = u.astype(jnp.int32)
  i = i.astype(jnp.int32)
  edges = edges.astype(jnp.int32)
  src = edges[0]
  dst = edges[1]
  zeros128 = jnp.zeros((NI, D), jnp.float32)
  ones128 = jnp.ones((CH, D), jnp.float32)

  deg0, deg1 = _deg_kernel(dst, zeros128, ones128)
  y1, dinv16 = _tc1(item_table, deg0, deg1, W1)
  s0, s1 = _prop_kernel(y1, src, dst, zeros128)
  y2 = _tc2(s0, s1, y1, dinv16, W2, b1.reshape(1, D))
  t0, t1 = _prop_kernel(y2, src, dst, zeros128)
  out = _score_kernel(u, i, user_table, t0, t1, y2, dinv16, b2)
  return out


# speedup vs baseline: 31.6571x; 1.3618x over previous
import functools
import jax
import jax.numpy as jnp
from jax import lax
from jax.experimental import pallas as pl
from jax.experimental.pallas import tpu as pltpu
from jax.experimental.pallas import tpu_sc as plsc

NI = 10000
D = 128
E = 320000
B = 4096
NC = 2
NS = 16
NW = NC * NS
EPT = E // NW
CH = 80
NCHUNK = EPT // CH
RPT = 632
RPT_LAST = NI - 15 * RPT
BPT = B // NW

_MESH = plsc.VectorSubcoreMesh(
    core_axis_name="c", subcore_axis_name="s", num_cores=NC, num_subcores=NS)


def _wid():
  return lax.axis_index("c") * NS + lax.axis_index("s")


def _copy_rows(src, dst, s):
  base = pl.multiple_of(s * RPT, 8)

  @pl.when(s < NS - 1)
  def _():
    pltpu.sync_copy(src.at[pl.ds(base, RPT)], dst.at[pl.ds(base, RPT)])

  @pl.when(s == NS - 1)
  def _():
    pltpu.sync_copy(src.at[pl.ds(base, RPT_LAST)], dst.at[pl.ds(base, RPT_LAST)])


DEG_RPT = 640
DEG_RPT_LAST = NI - 15 * DEG_RPT


@functools.partial(
    pl.kernel,
    out_type=[jax.ShapeDtypeStruct((NI, 16), jnp.float32),
              jax.ShapeDtypeStruct((NI, 16), jnp.float32)],
    mesh=_MESH,
    compiler_params=pltpu.CompilerParams(needs_layout_passes=False),
    scratch_types=[
        pltpu.VMEM((EPT,), jnp.int32),
        pltpu.VMEM((NI,), jnp.float32),
        pltpu.VMEM((NS * DEG_RPT,), jnp.float32),
        pltpu.VMEM((DEG_RPT,), jnp.float32),
        pltpu.VMEM((DEG_RPT, 16), jnp.float32),
        pltpu.VMEM_SHARED((NS * NI,), jnp.float32),
    ],
)
def _deg_kernel(dst_hbm, deg0_hbm, deg1_hbm,
                idx_v, hist_v, part_v, degvec_v, stage_v, part_sh):
  c = lax.axis_index("c")
  s = lax.axis_index("s")
  wid = c * NS + s
  pltpu.sync_copy(dst_hbm.at[pl.ds(pl.multiple_of(wid * EPT, 8), EPT)], idx_v)
  zero16 = jnp.zeros((16,), jnp.float32)
  one16 = jnp.ones((16,), jnp.float32)

  def zero_step(m, carry):
    hist_v[pl.ds(pl.multiple_of(16 * m, 16), 16)] = zero16
    return carry

  lax.fori_loop(0, NI // 16, zero_step, 0)

  def hist_step(m, carry):
    idxv = idx_v[pl.ds(pl.multiple_of(16 * m, 16), 16)]
    plsc.addupdate_scatter(hist_v, [idxv], one16)
    return carry

  lax.fori_loop(0, EPT // 16, hist_step, 0)
  pltpu.sync_copy(hist_v, part_sh.at[pl.ds(pl.multiple_of(s * NI, 8), NI)])
  plsc.subcore_barrier()

  def combine(r0, cnt):
    for t in range(NS):
      pltpu.sync_copy(
          part_sh.at[pl.ds(pl.multiple_of(t * NI + r0, 8), cnt)],
          part_v.at[pl.ds(t * DEG_RPT, cnt)])

    def col_block(jj, carry):
      o = pl.multiple_of(16 * jj, 16)
      acc = zero16
      for t in range(NS):
        acc = acc + part_v[pl.ds(pl.multiple_of(t * DEG_RPT, 16) + o, 16)]
      degvec_v[pl.ds(o, 16)] = acc
      for l in range(16):
        idxc = jnp.zeros((16,), jnp.int32) + (16 * jj + l)
        bc = plsc.load_gather(degvec_v, [idxc])
        stage_v[16 * jj + l, pl.ds(0, 16)] = bc
      return carry

    lax.fori_loop(0, cnt // 16, col_block, 0)

    @pl.when(c == 0)
    def _():
      pltpu.sync_copy(stage_v.at[pl.ds(0, cnt)], deg0_hbm.at[pl.ds(r0, cnt)])

    @pl.when(c == 1)
    def _():
      pltpu.sync_copy(stage_v.at[pl.ds(0, cnt)], deg1_hbm.at[pl.ds(r0, cnt)])

  @pl.when(s < NS - 1)
  def _():
    combine(pl.multiple_of(s * DEG_RPT, 8), DEG_RPT)

  @pl.when(s == NS - 1)
  def _():
    combine(pl.multiple_of(15 * DEG_RPT, 8), DEG_RPT_LAST)


NRING = 3
NFULL = (NCHUNK - 2) // NRING


@functools.partial(
    pl.kernel,
    out_type=[jax.ShapeDtypeStruct((NI, D), jnp.float32),
              jax.ShapeDtypeStruct((NI, D), jnp.float32)],
    mesh=_MESH,
    compiler_params=pltpu.CompilerParams(needs_layout_passes=False),
    scratch_types=[
        pltpu.VMEM((EPT,), jnp.int32),
        pltpu.VMEM((EPT,), jnp.int32),
        pltpu.VMEM((NRING, CH, D), jnp.float32),
        pltpu.VMEM_SHARED((NI, D), jnp.float32),
        pltpu.SemaphoreType.DMA,
        pltpu.SemaphoreType.DMA,
    ],
)
def _prop_kernel(y_hbm, src_hbm, dst_hbm, zeros_hbm, s0_hbm, s1_hbm,
                 sidx_v, didx_v, rows_v, acc_sh, gsem, ssem):
  c = lax.axis_index("c")
  s = lax.axis_index("s")
  wid = c * NS + s
  _copy_rows(zeros_hbm, acc_sh, s)
  base = pl.multiple_of(wid * EPT, 8)
  pltpu.sync_copy(src_hbm.at[pl.ds(base, EPT)], sidx_v)
  pltpu.sync_copy(dst_hbm.at[pl.ds(base, EPT)], didx_v)
  plsc.subcore_barrier()

  def chunk_slice(g):
    return pl.ds(pl.multiple_of(g * CH, 8), CH)

  def fire_gather(g, b):
    return pltpu.async_copy(y_hbm.at[sidx_v.at[chunk_slice(g)]],
                            rows_v.at[b], gsem)

  def wait_one_gather():
    pltpu.make_async_copy(zeros_hbm.at[pl.ds(0, CH)], rows_v.at[0],
                          gsem).wait()

  def fire_scatter(g, b):
    pltpu.async_copy(rows_v.at[b], acc_sh.at[didx_v.at[chunk_slice(g)]],
                     ssem, add=True)

  def drain_scatters(k):
    for _ in range(k):
      pltpu.make_async_copy(zeros_hbm.at[pl.ds(0, CH)], rows_v.at[0],
                            ssem).wait()

  def ring_round(k, carry):
    for j in range(NRING):
      g = k * NRING + j

      @pl.when(k > 0)
      def _():
        drain_scatters(1)
      fire_gather(g, j)
      if j == NRING - 1:
        wait_one_gather()
        fire_scatter(g - 2, (j + 1) % NRING)
      else:
        @pl.when(k > 0)
        def _():
          wait_one_gather()
          fire_scatter(g - 2, (j + 1) % NRING)
    return carry

  lax.fori_loop(0, NFULL, ring_round, 0)
  gl = NFULL * NRING
  wait_one_gather()
  fire_scatter(gl - 2, (gl - 2) % NRING)
  wait_one_gather()
  fire_scatter(gl - 1, (gl - 1) % NRING)
  drain_scatters(NRING)
  fire_gather(gl, gl % NRING)
  fire_gather(gl + 1, (gl + 1) % NRING)
  wait_one_gather()
  fire_scatter(gl, gl % NRING)
  wait_one_gather()
  fire_scatter(gl + 1, (gl + 1) % NRING)
  drain_scatters(2)
  plsc.subcore_barrier()

  @pl.when(c == 0)
  def _():
    _copy_rows(acc_sh, s0_hbm, s)

  @pl.when(c == 1)
  def _():
    _copy_rows(acc_sh, s1_hbm, s)


_RB = 1000


def _tc1_body(item_ref, deg0_ref, deg1_ref, w1_ref, y1_ref, dinv_ref):
  x = item_ref[...]
  ss = jnp.sum(x * x, axis=1, keepdims=True)
  n = jnp.sqrt(ss)
  scale = jnp.where(n > 1.0, 1.0 / (n + 1e-7), 1.0)
  deg = deg0_ref[:, 0:1] + deg1_ref[:, 0:1] + 1.0
  dinv = lax.rsqrt(deg)
  xs = x * (scale * dinv)
  y1_ref[...] = jnp.dot(xs, w1_ref[...], preferred_element_type=jnp.float32)
  dinv_ref[...] = jnp.broadcast_to(dinv, (_RB, D))


def _tc1(item_table, deg0, deg1, w1):
  return pl.pallas_call(
      _tc1_body,
      grid=(NI // _RB,),
      in_specs=[
          pl.BlockSpec((_RB, D), lambda g: (g, 0)),
          pl.BlockSpec((_RB, 16), lambda g: (g, 0)),
          pl.BlockSpec((_RB, 16), lambda g: (g, 0)),
          pl.BlockSpec((D, D), lambda g: (0, 0)),
      ],
      out_specs=[
          pl.BlockSpec((_RB, D), lambda g: (g, 0)),
          pl.BlockSpec((_RB, D), lambda g: (g, 0)),
      ],
      out_shape=[jax.ShapeDtypeStruct((NI, D), jnp.float32),
                 jax.ShapeDtypeStruct((NI, D), jnp.float32)],
  )(item_table, deg0, deg1, w1)


def _tc2_body(s0_ref, s1_ref, y1_ref, dinv_ref, w2_ref, b1_ref, y2_ref):
  dinv = dinv_ref[:, 0:1]
  z = dinv * (s0_ref[...] + s1_ref[...] + y1_ref[...]) + b1_ref[...]
  z = jnp.maximum(z, 0.0)
  y2_ref[...] = jnp.dot(z * dinv, w2_ref[...],
                        preferred_element_type=jnp.float32)


def _tc2(s0, s1, y1, dinv16, w2, b1):
  return pl.pallas_call(
      _tc2_body,
      grid=(NI // _RB,),
      in_specs=[
          pl.BlockSpec((_RB, D), lambda g: (g, 0)),
          pl.BlockSpec((_RB, D), lambda g: (g, 0)),
          pl.BlockSpec((_RB, D), lambda g: (g, 0)),
          pl.BlockSpec((_RB, D), lambda g: (g, 0)),
          pl.BlockSpec((D, D), lambda g: (0, 0)),
          pl.BlockSpec((1, D), lambda g: (0, 0)),
      ],
      out_specs=pl.BlockSpec((_RB, D), lambda g: (g, 0)),
      out_shape=jax.ShapeDtypeStruct((NI, D), jnp.float32),
  )(s0, s1, y1, dinv16, w2, b1)


def _rsqrt_nr(x):
  i = plsc.bitcast(x, jnp.int32)
  i = 0x5F3759DF - lax.shift_right_arithmetic(i, 1)
  y = plsc.bitcast(i, jnp.float32)
  for _ in range(3):
    y = y * (1.5 - 0.5 * x * y * y)
  return y


@functools.partial(
    pl.kernel,
    out_type=jax.ShapeDtypeStruct((B,), jnp.float32),
    mesh=_MESH,
    compiler_params=pltpu.CompilerParams(needs_layout_passes=False),
    scratch_types=[
        pltpu.VMEM((BPT,), jnp.int32),
        pltpu.VMEM((BPT,), jnp.int32),
        pltpu.VMEM((BPT, D), jnp.float32),
        pltpu.VMEM((BPT, D), jnp.float32),
        pltpu.VMEM((BPT, D), jnp.float32),
        pltpu.VMEM((BPT, D), jnp.float32),
        pltpu.VMEM((BPT, D), jnp.float32),
        pltpu.VMEM((D,), jnp.float32),
        pltpu.VMEM((BPT,), jnp.float32),
        pltpu.VMEM((256,), jnp.float32),
        pltpu.VMEM((256,), jnp.float32),
        pltpu.SemaphoreType.DMA,
    ],
)
def _score_kernel(u_hbm, i_hbm, utab_hbm, t0_hbm, t1_hbm, y2_hbm, dinv_hbm,
                  b2_hbm, out_hbm,
                  uidx_v, iidx_v, urows_v, t0_v, t1_v, y2_v, dv_v, b2_v,
                  res_v, ssm_v, dotm_v, sem):
  wid = _wid()
  base = pl.multiple_of(wid * BPT, 8)
  pltpu.sync_copy(u_hbm.at[pl.ds(base, BPT)], uidx_v)
  pltpu.sync_copy(i_hbm.at[pl.ds(base, BPT)], iidx_v)
  pltpu.sync_copy(b2_hbm, b2_v)
  pltpu.async_copy(utab_hbm.at[uidx_v], urows_v, sem).wait()
  pltpu.async_copy(t0_hbm.at[iidx_v], t0_v, sem).wait()
  pltpu.async_copy(t1_hbm.at[iidx_v], t1_v, sem).wait()
  pltpu.async_copy(y2_hbm.at[iidx_v], y2_v, sem).wait()
  pltpu.async_copy(dinv_hbm.at[iidx_v], dv_v, sem).wait()

  lane = lax.broadcasted_iota(jnp.int32, (16,), 0)

  def group(grp, carry):
    for j in range(16):
      r = grp * 16 + j
      dinv = dv_v[r, pl.ds(0, 16)]
      acc_ss = jnp.zeros((16,), jnp.float32)
      acc_dot = jnp.zeros((16,), jnp.float32)
      for k in range(D // 16):
        sl = pl.ds(16 * k, 16)
        uc = urows_v[r, sl]
        ic = dinv * (t0_v[r, sl] + t1_v[r, sl] + y2_v[r, sl]) + b2_v[sl]
        acc_ss = acc_ss + uc * uc
        acc_dot = acc_dot + uc * ic
      ssm_v[pl.ds(16 * j, 16)] = acc_ss
      dotm_v[pl.ds(16 * j, 16)] = acc_dot
    ss = jnp.zeros((16,), jnp.float32)
    dot = jnp.zeros((16,), jnp.float32)
    for k in range(16):
      col = lane * 16 + k
      ss = ss + plsc.load_gather(ssm_v, [col])
      dot = dot + plsc.load_gather(dotm_v, [col])
    rinv = _rsqrt_nr(jnp.maximum(ss, 1e-12))
    n = ss * rinv
    scale = jnp.where(n > 1.0, 1.0 / (n + 1e-7), 1.0)
    uv = scale * dot
    sig = 1.0 / (1.0 + jnp.exp(-uv))
    res_v[pl.ds(pl.multiple_of(grp * 16, 16), 16)] = sig
    return carry

  lax.fori_loop(0, BPT // 16, group, 0)
  pltpu.sync_copy(res_v, out_hbm.at[pl.ds(base, BPT)])


def kernel(u, i, edges, user_table, item_table, W1, b1, W2, b2):
  u = u.astype(jnp.int32)
  i = i.astype(jnp.int32)
  edges = edges.astype(jnp.int32)
  src = edges[0]
  dst = edges[1]
  zeros128 = jnp.zeros((NI, D), jnp.float32)

  deg0, deg1 = _deg_kernel(dst)
  y1, dinv16 = _tc1(item_table, deg0, deg1, W1)
  s0, s1 = _prop_kernel(y1, src, dst, zeros128)
  y2 = _tc2(s0, s1, y1, dinv16, W2, b1.reshape(1, D))
  t0, t1 = _prop_kernel(y2, src, dst, zeros128)
  out = _score_kernel(u, i, user_table, t0, t1, y2, dinv16, b2)
  return out


# speedup vs baseline: 31.9971x; 1.0107x over previous
import functools
import jax
import jax.numpy as jnp
from jax import lax
from jax.experimental import pallas as pl
from jax.experimental.pallas import tpu as pltpu
from jax.experimental.pallas import tpu_sc as plsc

NI = 10000
D = 128
E = 320000
B = 4096
NC = 2
NS = 16
NW = NC * NS
EPT = E // NW
CH = 80
NCHUNK = EPT // CH
RPT = 632
RPT_LAST = NI - 15 * RPT
BPT = B // NW

_MESH = plsc.VectorSubcoreMesh(
    core_axis_name="c", subcore_axis_name="s", num_cores=NC, num_subcores=NS)


def _wid():
  return lax.axis_index("c") * NS + lax.axis_index("s")


def _copy_rows(src, dst, s):
  base = pl.multiple_of(s * RPT, 8)

  @pl.when(s < NS - 1)
  def _():
    pltpu.sync_copy(src.at[pl.ds(base, RPT)], dst.at[pl.ds(base, RPT)])

  @pl.when(s == NS - 1)
  def _():
    pltpu.sync_copy(src.at[pl.ds(base, RPT_LAST)], dst.at[pl.ds(base, RPT_LAST)])


DEG_RPT = 640
DEG_RPT_LAST = NI - 15 * DEG_RPT


@functools.partial(
    pl.kernel,
    out_type=[jax.ShapeDtypeStruct((NI, 16), jnp.float32),
              jax.ShapeDtypeStruct((NI, 16), jnp.float32)],
    mesh=_MESH,
    compiler_params=pltpu.CompilerParams(needs_layout_passes=False),
    scratch_types=[
        pltpu.VMEM((EPT,), jnp.int32),
        pltpu.VMEM((NI,), jnp.float32),
        pltpu.VMEM((NS * DEG_RPT,), jnp.float32),
        pltpu.VMEM((DEG_RPT,), jnp.float32),
        pltpu.VMEM((DEG_RPT, 16), jnp.float32),
        pltpu.VMEM_SHARED((NS * NI,), jnp.float32),
    ],
)
def _deg_kernel(dst_hbm, deg0_hbm, deg1_hbm,
                idx_v, hist_v, part_v, degvec_v, stage_v, part_sh):
  c = lax.axis_index("c")
  s = lax.axis_index("s")
  wid = c * NS + s
  zero16 = jnp.zeros((16,), jnp.float32)
  one16 = jnp.ones((16,), jnp.float32)

  def zero_step(m, carry):
    hist_v[pl.ds(pl.multiple_of(16 * m, 16), 16)] = zero16
    return carry

  lax.fori_loop(0, NI // 16, zero_step, 0)
  pltpu.sync_copy(dst_hbm.at[pl.ds(pl.multiple_of(wid * EPT, 8), EPT)], idx_v)

  def hist_step(m, carry):
    idxv = idx_v[pl.ds(pl.multiple_of(16 * m, 16), 16)]
    plsc.addupdate_scatter(hist_v, [idxv], one16)
    return carry

  lax.fori_loop(0, EPT // 16, hist_step, 0)
  pltpu.sync_copy(hist_v, part_sh.at[pl.ds(pl.multiple_of(s * NI, 8), NI)])
  plsc.subcore_barrier()

  def combine(r0, cnt):
    for t in range(NS):
      pltpu.sync_copy(
          part_sh.at[pl.ds(pl.multiple_of(t * NI + r0, 8), cnt)],
          part_v.at[pl.ds(t * DEG_RPT, cnt)])

    def col_block(jj, carry):
      o = pl.multiple_of(16 * jj, 16)
      acc = zero16
      for t in range(NS):
        acc = acc + part_v[pl.ds(pl.multiple_of(t * DEG_RPT, 16) + o, 16)]
      degvec_v[pl.ds(o, 16)] = acc
      for l in range(16):
        idxc = jnp.zeros((16,), jnp.int32) + (16 * jj + l)
        bc = plsc.load_gather(degvec_v, [idxc])
        stage_v[16 * jj + l, pl.ds(0, 16)] = bc
      return carry

    lax.fori_loop(0, cnt // 16, col_block, 0)

    @pl.when(c == 0)
    def _():
      pltpu.sync_copy(stage_v.at[pl.ds(0, cnt)], deg0_hbm.at[pl.ds(r0, cnt)])

    @pl.when(c == 1)
    def _():
      pltpu.sync_copy(stage_v.at[pl.ds(0, cnt)], deg1_hbm.at[pl.ds(r0, cnt)])

  @pl.when(s < NS - 1)
  def _():
    combine(pl.multiple_of(s * DEG_RPT, 8), DEG_RPT)

  @pl.when(s == NS - 1)
  def _():
    combine(pl.multiple_of(15 * DEG_RPT, 8), DEG_RPT_LAST)


NRING = 3
NFULL = (NCHUNK - 2) // NRING


@functools.partial(
    pl.kernel,
    out_type=[jax.ShapeDtypeStruct((NI, D), jnp.float32),
              jax.ShapeDtypeStruct((NI, D), jnp.float32)],
    mesh=_MESH,
    compiler_params=pltpu.CompilerParams(needs_layout_passes=False),
    scratch_types=[
        pltpu.VMEM((EPT,), jnp.int32),
        pltpu.VMEM((EPT,), jnp.int32),
        pltpu.VMEM((NRING, CH, D), jnp.float32),
        pltpu.VMEM_SHARED((NI, D), jnp.float32),
        pltpu.SemaphoreType.DMA,
        pltpu.SemaphoreType.DMA,
    ],
)
def _prop_kernel(y_hbm, src_hbm, dst_hbm, zeros_hbm, s0_hbm, s1_hbm,
                 sidx_v, didx_v, rows_v, acc_sh, gsem, ssem):
  c = lax.axis_index("c")
  s = lax.axis_index("s")
  wid = c * NS + s
  base = pl.multiple_of(wid * EPT, 8)
  d1 = pltpu.async_copy(src_hbm.at[pl.ds(base, EPT)], sidx_v, gsem)
  d2 = pltpu.async_copy(dst_hbm.at[pl.ds(base, EPT)], didx_v, gsem)
  _copy_rows(zeros_hbm, acc_sh, s)
  d1.wait()
  d2.wait()
  plsc.subcore_barrier()

  def chunk_slice(g):
    return pl.ds(pl.multiple_of(g * CH, 8), CH)

  def fire_gather(g, b):
    return pltpu.async_copy(y_hbm.at[sidx_v.at[chunk_slice(g)]],
                            rows_v.at[b], gsem)

  def wait_one_gather():
    pltpu.make_async_copy(zeros_hbm.at[pl.ds(0, CH)], rows_v.at[0],
                          gsem).wait()

  def fire_scatter(g, b):
    pltpu.async_copy(rows_v.at[b], acc_sh.at[didx_v.at[chunk_slice(g)]],
                     ssem, add=True)

  def drain_scatters(k):
    for _ in range(k):
      pltpu.make_async_copy(zeros_hbm.at[pl.ds(0, CH)], rows_v.at[0],
                            ssem).wait()

  def ring_round(k, carry):
    for j in range(NRING):
      g = k * NRING + j

      @pl.when(k > 0)
      def _():
        drain_scatters(1)
      fire_gather(g, j)
      if j == NRING - 1:
        wait_one_gather()
        fire_scatter(g - 2, (j + 1) % NRING)
      else:
        @pl.when(k > 0)
        def _():
          wait_one_gather()
          fire_scatter(g - 2, (j + 1) % NRING)
    return carry

  lax.fori_loop(0, NFULL, ring_round, 0)
  gl = NFULL * NRING
  wait_one_gather()
  fire_scatter(gl - 2, (gl - 2) % NRING)
  wait_one_gather()
  fire_scatter(gl - 1, (gl - 1) % NRING)
  drain_scatters(NRING)
  fire_gather(gl, gl % NRING)
  fire_gather(gl + 1, (gl + 1) % NRING)
  wait_one_gather()
  fire_scatter(gl, gl % NRING)
  wait_one_gather()
  fire_scatter(gl + 1, (gl + 1) % NRING)
  drain_scatters(2)
  plsc.subcore_barrier()

  @pl.when(c == 0)
  def _():
    _copy_rows(acc_sh, s0_hbm, s)

  @pl.when(c == 1)
  def _():
    _copy_rows(acc_sh, s1_hbm, s)


_RB = 1000


def _tc1_body(item_ref, deg0_ref, deg1_ref, w1_ref, y1_ref, dinv_ref):
  x = item_ref[...]
  ss = jnp.sum(x * x, axis=1, keepdims=True)
  n = jnp.sqrt(ss)
  scale = jnp.where(n > 1.0, 1.0 / (n + 1e-7), 1.0)
  deg = deg0_ref[:, 0:1] + deg1_ref[:, 0:1] + 1.0
  dinv = lax.rsqrt(deg)
  xs = x * (scale * dinv)
  y1_ref[...] = jnp.dot(xs, w1_ref[...], preferred_element_type=jnp.float32)
  dinv_ref[...] = jnp.broadcast_to(dinv, (_RB, D))


def _tc1(item_table, deg0, deg1, w1):
  return pl.pallas_call(
      _tc1_body,
      grid=(NI // _RB,),
      in_specs=[
          pl.BlockSpec((_RB, D), lambda g: (g, 0)),
          pl.BlockSpec((_RB, 16), lambda g: (g, 0)),
          pl.BlockSpec((_RB, 16), lambda g: (g, 0)),
          pl.BlockSpec((D, D), lambda g: (0, 0)),
      ],
      out_specs=[
          pl.BlockSpec((_RB, D), lambda g: (g, 0)),
          pl.BlockSpec((_RB, D), lambda g: (g, 0)),
      ],
      out_shape=[jax.ShapeDtypeStruct((NI, D), jnp.float32),
                 jax.ShapeDtypeStruct((NI, D), jnp.float32)],
  )(item_table, deg0, deg1, w1)


def _tc2_body(s0_ref, s1_ref, y1_ref, dinv_ref, w2_ref, b1_ref, y2_ref):
  dinv = dinv_ref[:, 0:1]
  z = dinv * (s0_ref[...] + s1_ref[...] + y1_ref[...]) + b1_ref[...]
  z = jnp.maximum(z, 0.0)
  y2_ref[...] = jnp.dot(z * dinv, w2_ref[...],
                        preferred_element_type=jnp.float32)


def _tc2(s0, s1, y1, dinv16, w2, b1):
  return pl.pallas_call(
      _tc2_body,
      grid=(NI // _RB,),
      in_specs=[
          pl.BlockSpec((_RB, D), lambda g: (g, 0)),
          pl.BlockSpec((_RB, D), lambda g: (g, 0)),
          pl.BlockSpec((_RB, D), lambda g: (g, 0)),
          pl.BlockSpec((_RB, D), lambda g: (g, 0)),
          pl.BlockSpec((D, D), lambda g: (0, 0)),
          pl.BlockSpec((1, D), lambda g: (0, 0)),
      ],
      out_specs=pl.BlockSpec((_RB, D), lambda g: (g, 0)),
      out_shape=jax.ShapeDtypeStruct((NI, D), jnp.float32),
  )(s0, s1, y1, dinv16, w2, b1)


def _rsqrt_nr(x):
  i = plsc.bitcast(x, jnp.int32)
  i = 0x5F3759DF - lax.shift_right_arithmetic(i, 1)
  y = plsc.bitcast(i, jnp.float32)
  for _ in range(3):
    y = y * (1.5 - 0.5 * x * y * y)
  return y


@functools.partial(
    pl.kernel,
    out_type=jax.ShapeDtypeStruct((B,), jnp.float32),
    mesh=_MESH,
    compiler_params=pltpu.CompilerParams(needs_layout_passes=False),
    scratch_types=[
        pltpu.VMEM((BPT,), jnp.int32),
        pltpu.VMEM((BPT,), jnp.int32),
        pltpu.VMEM((BPT, D), jnp.float32),
        pltpu.VMEM((BPT, D), jnp.float32),
        pltpu.VMEM((BPT, D), jnp.float32),
        pltpu.VMEM((BPT, D), jnp.float32),
        pltpu.VMEM((BPT, D), jnp.float32),
        pltpu.VMEM((D,), jnp.float32),
        pltpu.VMEM((BPT,), jnp.float32),
        pltpu.VMEM((256,), jnp.float32),
        pltpu.VMEM((256,), jnp.float32),
        pltpu.SemaphoreType.DMA,
    ],
)
def _score_kernel(u_hbm, i_hbm, utab_hbm, t0_hbm, t1_hbm, y2_hbm, dinv_hbm,
                  b2_hbm, out_hbm,
                  uidx_v, iidx_v, urows_v, t0_v, t1_v, y2_v, dv_v, b2_v,
                  res_v, ssm_v, dotm_v, sem):
  wid = _wid()
  base = pl.multiple_of(wid * BPT, 8)
  pltpu.sync_copy(u_hbm.at[pl.ds(base, BPT)], uidx_v)
  pltpu.sync_copy(i_hbm.at[pl.ds(base, BPT)], iidx_v)
  pltpu.sync_copy(b2_hbm, b2_v)
  pltpu.async_copy(utab_hbm.at[uidx_v], urows_v, sem).wait()
  pltpu.async_copy(t0_hbm.at[iidx_v], t0_v, sem).wait()
  pltpu.async_copy(t1_hbm.at[iidx_v], t1_v, sem).wait()
  pltpu.async_copy(y2_hbm.at[iidx_v], y2_v, sem).wait()
  pltpu.async_copy(dinv_hbm.at[iidx_v], dv_v, sem).wait()

  lane = lax.broadcasted_iota(jnp.int32, (16,), 0)

  def group(grp, carry):
    for j in range(16):
      r = grp * 16 + j
      dinv = dv_v[r, pl.ds(0, 16)]
      acc_ss = jnp.zeros((16,), jnp.float32)
      acc_dot = jnp.zeros((16,), jnp.float32)
      for k in range(D // 16):
        sl = pl.ds(16 * k, 16)
        uc = urows_v[r, sl]
        ic = dinv * (t0_v[r, sl] + t1_v[r, sl] + y2_v[r, sl]) + b2_v[sl]
        acc_ss = acc_ss + uc * uc
        acc_dot = acc_dot + uc * ic
      ssm_v[pl.ds(16 * j, 16)] = acc_ss
      dotm_v[pl.ds(16 * j, 16)] = acc_dot
    ss = jnp.zeros((16,), jnp.float32)
    dot = jnp.zeros((16,), jnp.float32)
    for k in range(16):
      col = lane * 16 + k
      ss = ss + plsc.load_gather(ssm_v, [col])
      dot = dot + plsc.load_gather(dotm_v, [col])
    rinv = _rsqrt_nr(jnp.maximum(ss, 1e-12))
    n = ss * rinv
    scale = jnp.where(n > 1.0, 1.0 / (n + 1e-7), 1.0)
    uv = scale * dot
    sig = 1.0 / (1.0 + jnp.exp(-uv))
    res_v[pl.ds(pl.multiple_of(grp * 16, 16), 16)] = sig
    return carry

  lax.fori_loop(0, BPT // 16, group, 0)
  pltpu.sync_copy(res_v, out_hbm.at[pl.ds(base, BPT)])


def kernel(u, i, edges, user_table, item_table, W1, b1, W2, b2):
  u = u.astype(jnp.int32)
  i = i.astype(jnp.int32)
  edges = edges.astype(jnp.int32)
  src = edges[0]
  dst = edges[1]
  zeros128 = jnp.zeros((NI, D), jnp.float32)

  deg0, deg1 = _deg_kernel(dst)
  y1, dinv16 = _tc1(item_table, deg0, deg1, W1)
  s0, s1 = _prop_kernel(y1, src, dst, zeros128)
  y2 = _tc2(s0, s1, y1, dinv16, W2, b1.reshape(1, D))
  t0, t1 = _prop_kernel(y2, src, dst, zeros128)
  out = _score_kernel(u, i, user_table, t0, t1, y2, dinv16, b2)
  return out
